# baseline (speedup 1.0000x reference)
import jax
import jax.numpy as jnp
from jax.experimental import pallas as pl


def _onehot_body(idx_ref, out_ref):
    bb, n, m = out_ref.shape
    iota_n = jax.lax.broadcasted_iota(jnp.int32, (bb, n, m), 1)
    out_ref[...] = idx_ref[...] == iota_n


def kernel(x, w, idx):
    B, N = x.shape
    M = w.shape[1]
    idx32 = idx.astype(jnp.int32)
    BB = 256
    out = pl.pallas_call(
        _onehot_body,
        grid=(B // BB,),
        in_specs=[pl.BlockSpec((BB, 1, M), lambda i: (i, 0, 0))],
        out_specs=pl.BlockSpec((BB, N, M), lambda i: (i, 0, 0)),
        out_shape=jax.ShapeDtypeStruct((B, N, M), jnp.bool_),
    )(idx32)
    return out


# speedup vs baseline: 1.0011x; 1.0011x over previous
import jax
import jax.numpy as jnp
from jax.experimental import pallas as pl


def _onehot_body(idx_ref, out_ref):
    bb, n, m = out_ref.shape
    iota_n = jax.lax.broadcasted_iota(jnp.int32, (bb, n, m), 1)
    out_ref[...] = idx_ref[...] == iota_n


def kernel(x, w, idx):
    B, N = x.shape
    M = w.shape[1]
    idx32 = idx.astype(jnp.int32)
    BB = 32
    out = pl.pallas_call(
        _onehot_body,
        grid=(B // BB,),
        in_specs=[pl.BlockSpec((BB, 1, M), lambda i: (i, 0, 0))],
        out_specs=pl.BlockSpec((BB, N, M), lambda i: (i, 0, 0)),
        out_shape=jax.ShapeDtypeStruct((B, N, M), jnp.bool_),
    )(idx32)
    return out


# speedup vs baseline: 1.7927x; 1.7908x over previous
import jax
import jax.numpy as jnp
from jax.experimental import pallas as pl


def _onehot_body(idx_ref, out_ref):
    bb, n, m = out_ref.shape
    iota_n = jax.lax.broadcasted_iota(jnp.int32, (bb, n, m), 1)
    out_ref[...] = (idx_ref[...] == iota_n).astype(jnp.int8)


def kernel(x, w, idx):
    B, N = x.shape
    M = w.shape[1]
    idx32 = idx.astype(jnp.int32)
    BB = 64
    out = pl.pallas_call(
        _onehot_body,
        grid=(B // BB,),
        in_specs=[pl.BlockSpec((BB, 1, M), lambda i: (i, 0, 0))],
        out_specs=pl.BlockSpec((BB, N, M), lambda i: (i, 0, 0)),
        out_shape=jax.ShapeDtypeStruct((B, N, M), jnp.int8),
    )(idx32)
    return out.astype(jnp.bool_)
